# Initial kernel scaffold; baseline (speedup 1.0000x reference)
import jax
import jax.numpy as jnp
from jax.experimental import pallas as pl


def kernel(x, edge_index, W1, b1, W2, b2):
    raise NotImplementedError("write your pallas kernel here")


# speedup vs baseline: 27.7098x; 27.7098x over previous
import functools

import jax
import jax.numpy as jnp
from jax import lax
from jax.experimental import pallas as pl
from jax.experimental.pallas import tpu as pltpu
from jax.experimental.pallas import tpu_sc as plsc

NC = 2
NS = 16
L = 16
NW = NC * NS
CHUNK = 128
NBUF = 4
F = 16


def _sc_mesh():
  return plsc.VectorSubcoreMesh(
      core_axis_name="c", subcore_axis_name="s", num_cores=NC,
      num_subcores=NS)


def _make_sc_agg(n_pad, n_chunks):
  rows_per_tile = n_pad // NS
  n_outer = n_chunks // NBUF

  def body(y_hbm, cols_hbm, rows_hbm, out_hbm, col_v, row_v, bufs, acc,
           gsem):
    c = lax.axis_index("c")
    s = lax.axis_index("s")
    wid = c * NS + s

    zvec = jnp.zeros((L,), jnp.float32)

    @pl.loop(0, CHUNK)
    def _(i):
      bufs[0, i, :] = zvec

    @pl.loop(0, rows_per_tile // CHUNK)
    def _(jz):
      pltpu.sync_copy(bufs.at[0],
                      acc.at[pl.ds(s * rows_per_tile + jz * CHUNK, CHUNK)])

    pltpu.sync_copy(cols_hbm.at[wid], col_v)
    pltpu.sync_copy(rows_hbm.at[wid], row_v)

    for b in range(NBUF):
      pltpu.async_copy(y_hbm.at[col_v.at[b]], bufs.at[b], gsem)

    plsc.subcore_barrier()

    @pl.loop(0, n_outer - 1)
    def _(g):
      base = g * NBUF
      for b in range(NBUF):
        j = base + b
        pltpu.make_async_copy(y_hbm.at[col_v.at[j]], bufs.at[b],
                              gsem).wait()
        pltpu.sync_copy(bufs.at[b], acc.at[row_v.at[j]], add=True)
        pltpu.async_copy(y_hbm.at[col_v.at[j + NBUF]], bufs.at[b], gsem)

    for b in range(NBUF):
      j = (n_outer - 1) * NBUF + b
      pltpu.make_async_copy(y_hbm.at[col_v.at[j]], bufs.at[b], gsem).wait()
      pltpu.sync_copy(bufs.at[b], acc.at[row_v.at[j]], add=True)

    plsc.subcore_barrier()

    pltpu.sync_copy(
        acc.at[pl.ds(s * rows_per_tile, rows_per_tile)],
        out_hbm.at[c, pl.ds(s * rows_per_tile, rows_per_tile)])

  return pl.kernel(
      body,
      out_type=jax.ShapeDtypeStruct((NC, n_pad, F), jnp.float32),
      mesh=_sc_mesh(),
      compiler_params=pltpu.CompilerParams(use_tc_tiling_on_sc=False),
      scratch_types=[
          pltpu.VMEM((n_chunks, CHUNK), jnp.int32),
          pltpu.VMEM((n_chunks, CHUNK), jnp.int32),
          pltpu.VMEM((NBUF, CHUNK, F), jnp.float32),
          pltpu.VMEM_SHARED((n_pad, F), jnp.float32),
          pltpu.SemaphoreType.DMA,
      ],
  )


def _make_sc_deg(n_pad, n_chunks):
  rows_per_tile = n_pad // NS

  def body(rows_hbm, out_hbm, row_v, buf, acc):
    c = lax.axis_index("c")
    s = lax.axis_index("s")
    wid = c * NS + s

    zvec = jnp.zeros((L,), jnp.float32)

    @pl.loop(0, CHUNK)
    def _(i):
      buf[i, :] = zvec

    @pl.loop(0, rows_per_tile // CHUNK)
    def _(jz):
      pltpu.sync_copy(buf,
                      acc.at[pl.ds(s * rows_per_tile + jz * CHUNK, CHUNK)])

    pltpu.sync_copy(rows_hbm.at[wid], row_v)

    ovec = jnp.ones((L,), jnp.float32)

    @pl.loop(0, CHUNK)
    def _(i):
      buf[i, :] = ovec

    plsc.subcore_barrier()

    @pl.loop(0, n_chunks)
    def _(j):
      pltpu.sync_copy(buf, acc.at[row_v.at[j]], add=True)

    plsc.subcore_barrier()

    pltpu.sync_copy(
        acc.at[pl.ds(s * rows_per_tile, rows_per_tile)],
        out_hbm.at[c, pl.ds(s * rows_per_tile, rows_per_tile)])

  return pl.kernel(
      body,
      out_type=jax.ShapeDtypeStruct((NC, n_pad, F), jnp.float32),
      mesh=_sc_mesh(),
      compiler_params=pltpu.CompilerParams(use_tc_tiling_on_sc=False),
      scratch_types=[
          pltpu.VMEM((n_chunks, CHUNK), jnp.int32),
          pltpu.VMEM((CHUNK, F), jnp.float32),
          pltpu.VMEM_SHARED((n_pad, F), jnp.float32),
      ],
  )


_R = 2000


def _tc1_body(x, w10, w11, d0, d1, xw0_o, y1_o, dis_o):
  deg = d0[...] + d1[...]
  dis = jnp.where(deg > 0.0, lax.rsqrt(jnp.where(deg > 0.0, deg, 1.0)), 0.0)
  xv = x[...]
  xw0_o[...] = jnp.dot(xv, w10[...], preferred_element_type=jnp.float32)
  y1_o[...] = dis * jnp.dot(xv, w11[...], preferred_element_type=jnp.float32)
  dis_o[...] = dis


def _tc2_body(a0, a1, xw0, dis, b1, w20, w21, hw0_o, y2_o):
  h = jnp.maximum(xw0[...] - dis[...] * (a0[...] + a1[...]) + b1[...], 0.0)
  hw0_o[...] = jnp.dot(h, w20[...], preferred_element_type=jnp.float32)
  y2_o[...] = dis[...] * jnp.dot(h, w21[...],
                                 preferred_element_type=jnp.float32)


def _tc3_body(a0, a1, hw0, dis, b2, out_o):
  z = hw0[...] - dis[...] * (a0[...] + a1[...]) + b2[...]
  m = jnp.max(z, axis=1, keepdims=True)
  e = jnp.exp(z - m)
  out_o[...] = (z - m) - jnp.log(jnp.sum(e, axis=1, keepdims=True))


def _row_spec(w):
  return pl.BlockSpec((_R, w), lambda i: (i, 0))


def _full_spec(shape):
  return pl.BlockSpec(shape, lambda i: tuple(0 for _ in shape))


def kernel(x, edge_index, W1, b1, W2, b2):
  n, f_in = x.shape
  e = edge_index.shape[1]
  hid = W1.shape[2]
  c_out = W2.shape[2]

  per_w_chunks = -(-e // (NW * CHUNK))
  n_chunks = -(-per_w_chunks // NBUF) * NBUF
  e_pad = NW * n_chunks * CHUNK
  n_pad = -(-n // (NS * CHUNK)) * (NS * CHUNK)

  row = edge_index[0]
  col = edge_index[1]
  pad = jnp.full((e_pad - e,), n, jnp.int32)
  rows3 = jnp.concatenate([row, pad]).reshape(NW, n_chunks, CHUNK)
  cols3 = jnp.concatenate([col, pad]).reshape(NW, n_chunks, CHUNK)

  sc_deg = _make_sc_deg(n_pad, n_chunks)
  sc_agg = _make_sc_agg(n_pad, n_chunks)

  degp = sc_deg(rows3)
  d0 = degp[0, :n, :]
  d1 = degp[1, :n, :]

  grid = (n // _R,)
  xw0, y1, dis = pl.pallas_call(
      _tc1_body,
      grid=grid,
      in_specs=[
          _row_spec(f_in),
          _full_spec((f_in, hid)),
          _full_spec((f_in, hid)),
          _row_spec(F),
          _row_spec(F),
      ],
      out_specs=[_row_spec(hid), _row_spec(hid), _row_spec(F)],
      out_shape=[
          jax.ShapeDtypeStruct((n, hid), jnp.float32),
          jax.ShapeDtypeStruct((n, hid), jnp.float32),
          jax.ShapeDtypeStruct((n, F), jnp.float32),
      ],
  )(x, W1[0], W1[1], d0, d1)

  y1p = jnp.pad(y1, ((0, n_pad - n), (0, 0)))
  acc1 = sc_agg(y1p, cols3, rows3)

  hw0, y2 = pl.pallas_call(
      _tc2_body,
      grid=grid,
      in_specs=[
          _row_spec(hid),
          _row_spec(hid),
          _row_spec(hid),
          _row_spec(F),
          _full_spec((1, hid)),
          _full_spec((hid, c_out)),
          _full_spec((hid, c_out)),
      ],
      out_specs=[_row_spec(c_out), _row_spec(c_out)],
      out_shape=[
          jax.ShapeDtypeStruct((n, c_out), jnp.float32),
          jax.ShapeDtypeStruct((n, c_out), jnp.float32),
      ],
  )(acc1[0, :n, :], acc1[1, :n, :], xw0, dis, b1.reshape(1, hid), W2[0],
    W2[1])

  y2p = jnp.pad(y2, ((0, n_pad - n), (0, 0)))
  acc2 = sc_agg(y2p, cols3, rows3)

  out = pl.pallas_call(
      _tc3_body,
      grid=grid,
      in_specs=[
          _row_spec(c_out),
          _row_spec(c_out),
          _row_spec(c_out),
          _row_spec(F),
          _full_spec((1, c_out)),
      ],
      out_specs=_row_spec(c_out),
      out_shape=jax.ShapeDtypeStruct((n, c_out), jnp.float32),
  )(acc2[0, :n, :], acc2[1, :n, :], hw0, dis, b2.reshape(1, c_out))

  return out


# speedup vs baseline: 28.2728x; 1.0203x over previous
import functools

import jax
import jax.numpy as jnp
from jax import lax
from jax.experimental import pallas as pl
from jax.experimental.pallas import tpu as pltpu
from jax.experimental.pallas import tpu_sc as plsc

NC = 2
NS = 16
L = 16
NW = NC * NS
CHUNK = 128
GRP = 4
F = 16
DW = 8


def _sc_mesh():
  return plsc.VectorSubcoreMesh(
      core_axis_name="c", subcore_axis_name="s", num_cores=NC,
      num_subcores=NS)


def _make_sc_agg(n_pad, n_chunks):
  rows_per_tile = n_pad // NS
  n_groups = n_chunks // GRP

  def body(y_hbm, cols_hbm, rows_hbm, out_hbm, col_v, row_v, bufs, acc,
           gsems, ssems):
    c = lax.axis_index("c")
    s = lax.axis_index("s")
    wid = c * NS + s

    zvec = jnp.zeros((L,), jnp.float32)

    @pl.loop(0, CHUNK)
    def _(i):
      bufs[0, 0, i, :] = zvec

    @pl.loop(0, rows_per_tile // CHUNK)
    def _(jz):
      pltpu.sync_copy(bufs.at[0, 0],
                      acc.at[pl.ds(s * rows_per_tile + jz * CHUNK, CHUNK)])

    pltpu.sync_copy(cols_hbm.at[wid], col_v)
    pltpu.sync_copy(rows_hbm.at[wid], row_v)

    plsc.subcore_barrier()

    def gather(i):
      p = i % 2
      for b in range(GRP):
        j = i * GRP + b
        pltpu.async_copy(y_hbm.at[col_v.at[j]], bufs.at[p, b], gsems[p])

    gather(0)
    for i in range(n_groups):
      p = i % 2
      if i + 1 < n_groups:
        if i >= 1:
          for b in range(GRP):
            pltpu.make_async_copy(bufs.at[1 - p, b],
                                  acc.at[row_v.at[b]], ssems[1 - p]).wait()
        gather(i + 1)
      for b in range(GRP):
        j = i * GRP + b
        pltpu.make_async_copy(y_hbm.at[col_v.at[j]], bufs.at[p, b],
                              gsems[p]).wait()
      for b in range(GRP):
        j = i * GRP + b
        pltpu.async_copy(bufs.at[p, b], acc.at[row_v.at[j]], ssems[p],
                         add=True)
    for i in (n_groups - 2, n_groups - 1):
      p = i % 2
      for b in range(GRP):
        pltpu.make_async_copy(bufs.at[p, b], acc.at[row_v.at[b]],
                              ssems[p]).wait()

    plsc.subcore_barrier()

    pltpu.sync_copy(
        acc.at[pl.ds(s * rows_per_tile, rows_per_tile)],
        out_hbm.at[c, pl.ds(s * rows_per_tile, rows_per_tile)])

  return pl.kernel(
      body,
      out_type=jax.ShapeDtypeStruct((NC, n_pad, F), jnp.float32),
      mesh=_sc_mesh(),
      compiler_params=pltpu.CompilerParams(use_tc_tiling_on_sc=False),
      scratch_types=[
          pltpu.VMEM((n_chunks, CHUNK), jnp.int32),
          pltpu.VMEM((n_chunks, CHUNK), jnp.int32),
          pltpu.VMEM((2, GRP, CHUNK, F), jnp.float32),
          pltpu.VMEM_SHARED((n_pad, F), jnp.float32),
          [pltpu.SemaphoreType.DMA, pltpu.SemaphoreType.DMA],
          [pltpu.SemaphoreType.DMA, pltpu.SemaphoreType.DMA],
      ],
  )


def _make_sc_deg(n_pad, n_chunks):
  rows_per_tile = n_pad // NS
  DEPTH = 8

  def body(rows_hbm, zeros_hbm, ones_hbm, out_hbm, row_v, ones_v, acc,
           ssem):
    c = lax.axis_index("c")
    s = lax.axis_index("s")
    wid = c * NS + s

    pltpu.sync_copy(zeros_hbm,
                    acc.at[pl.ds(s * rows_per_tile, rows_per_tile)])
    pltpu.sync_copy(ones_hbm, ones_v)
    pltpu.sync_copy(rows_hbm.at[wid], row_v)

    plsc.subcore_barrier()

    @pl.loop(0, n_chunks // DEPTH)
    def _(g):
      for b in range(DEPTH):
        pltpu.async_copy(ones_v, acc.at[row_v.at[g * DEPTH + b]], ssem,
                         add=True)
      for b in range(DEPTH):
        pltpu.make_async_copy(ones_v, acc.at[row_v.at[b]], ssem).wait()

    plsc.subcore_barrier()

    pltpu.sync_copy(
        acc.at[pl.ds(s * rows_per_tile, rows_per_tile)],
        out_hbm.at[c, pl.ds(s * rows_per_tile, rows_per_tile)])

  return pl.kernel(
      body,
      out_type=jax.ShapeDtypeStruct((NC, n_pad, DW), jnp.float32),
      mesh=_sc_mesh(),
      compiler_params=pltpu.CompilerParams(use_tc_tiling_on_sc=False),
      scratch_types=[
          pltpu.VMEM((n_chunks, CHUNK), jnp.int32),
          pltpu.VMEM((CHUNK, DW), jnp.float32),
          pltpu.VMEM_SHARED((n_pad, DW), jnp.float32),
          pltpu.SemaphoreType.DMA,
      ],
  )


_R = 2000


def _tc1a_body(x, w10, w11, xw0_o, xw1_o):
  xv = x[...]
  xw0_o[...] = jnp.dot(xv, w10[...], preferred_element_type=jnp.float32)
  xw1_o[...] = jnp.dot(xv, w11[...], preferred_element_type=jnp.float32)


def _tc1b_body(d0, d1, xw1, y1_o, dis_o):
  deg = d0[...] + d1[...]
  dis8 = jnp.where(deg > 0.0, lax.rsqrt(jnp.where(deg > 0.0, deg, 1.0)),
                   0.0)
  dis = jnp.concatenate([dis8, dis8], axis=1)
  y1_o[...] = dis * xw1[...]
  dis_o[...] = dis


def _tc2_body(a0, a1, xw0, dis, b1, w20, w21, hw0_o, y2_o):
  h = jnp.maximum(xw0[...] - dis[...] * (a0[...] + a1[...]) + b1[...], 0.0)
  hw0_o[...] = jnp.dot(h, w20[...], preferred_element_type=jnp.float32)
  y2_o[...] = dis[...] * jnp.dot(h, w21[...],
                                 preferred_element_type=jnp.float32)


def _tc3_body(a0, a1, hw0, dis, b2, out_o):
  z = hw0[...] - dis[...] * (a0[...] + a1[...]) + b2[...]
  m = jnp.max(z, axis=1, keepdims=True)
  e = jnp.exp(z - m)
  out_o[...] = (z - m) - jnp.log(jnp.sum(e, axis=1, keepdims=True))


def _row_spec(w):
  return pl.BlockSpec((_R, w), lambda i: (i, 0))


def _full_spec(shape):
  return pl.BlockSpec(shape, lambda i: tuple(0 for _ in shape))


def kernel(x, edge_index, W1, b1, W2, b2):
  n, f_in = x.shape
  e = edge_index.shape[1]
  hid = W1.shape[2]
  c_out = W2.shape[2]

  per_w_chunks = -(-e // (NW * CHUNK))
  n_chunks = -(-per_w_chunks // (2 * GRP)) * (2 * GRP)
  e_pad = NW * n_chunks * CHUNK
  n_pad = -(-n // (NS * CHUNK)) * (NS * CHUNK)

  row = edge_index[0]
  col = edge_index[1]
  pad = jnp.full((e_pad - e,), n, jnp.int32)
  rows3 = jnp.concatenate([row, pad]).reshape(NW, n_chunks, CHUNK)
  cols3 = jnp.concatenate([col, pad]).reshape(NW, n_chunks, CHUNK)

  sc_deg = _make_sc_deg(n_pad, n_chunks)
  sc_agg = _make_sc_agg(n_pad, n_chunks)

  degp = sc_deg(rows3, jnp.zeros((n_pad // NS, DW), jnp.float32),
                jnp.ones((CHUNK, DW), jnp.float32))
  d0 = degp[0, :n, :]
  d1 = degp[1, :n, :]

  grid = (n // _R,)
  xw0, xw1 = pl.pallas_call(
      _tc1a_body,
      grid=grid,
      in_specs=[
          _row_spec(f_in),
          _full_spec((f_in, hid)),
          _full_spec((f_in, hid)),
      ],
      out_specs=[_row_spec(hid), _row_spec(hid)],
      out_shape=[
          jax.ShapeDtypeStruct((n, hid), jnp.float32),
          jax.ShapeDtypeStruct((n, hid), jnp.float32),
      ],
  )(x, W1[0], W1[1])

  y1, dis = pl.pallas_call(
      _tc1b_body,
      grid=grid,
      in_specs=[_row_spec(DW), _row_spec(DW), _row_spec(hid)],
      out_specs=[_row_spec(hid), _row_spec(F)],
      out_shape=[
          jax.ShapeDtypeStruct((n, hid), jnp.float32),
          jax.ShapeDtypeStruct((n, F), jnp.float32),
      ],
  )(d0, d1, xw1)

  y1p = jnp.pad(y1, ((0, n_pad - n), (0, 0)))
  acc1 = sc_agg(y1p, cols3, rows3)

  hw0, y2 = pl.pallas_call(
      _tc2_body,
      grid=grid,
      in_specs=[
          _row_spec(hid),
          _row_spec(hid),
          _row_spec(hid),
          _row_spec(F),
          _full_spec((1, hid)),
          _full_spec((hid, c_out)),
          _full_spec((hid, c_out)),
      ],
      out_specs=[_row_spec(c_out), _row_spec(c_out)],
      out_shape=[
          jax.ShapeDtypeStruct((n, c_out), jnp.float32),
          jax.ShapeDtypeStruct((n, c_out), jnp.float32),
      ],
  )(acc1[0, :n, :], acc1[1, :n, :], xw0, dis, b1.reshape(1, hid), W2[0],
    W2[1])

  y2p = jnp.pad(y2, ((0, n_pad - n), (0, 0)))
  acc2 = sc_agg(y2p, cols3, rows3)

  out = pl.pallas_call(
      _tc3_body,
      grid=grid,
      in_specs=[
          _row_spec(c_out),
          _row_spec(c_out),
          _row_spec(c_out),
          _row_spec(F),
          _full_spec((1, c_out)),
      ],
      out_specs=_row_spec(c_out),
      out_shape=jax.ShapeDtypeStruct((n, c_out), jnp.float32),
  )(acc2[0, :n, :], acc2[1, :n, :], hw0, dis, b2.reshape(1, c_out))

  return out


# speedup vs baseline: 29.0324x; 1.0269x over previous
import functools

import jax
import jax.numpy as jnp
from jax import lax
from jax.experimental import pallas as pl
from jax.experimental.pallas import tpu as pltpu
from jax.experimental.pallas import tpu_sc as plsc

NC = 1
NS = 16
L = 16
NW = NC * NS
CHUNK = 128
GRP = 4
F = 16
DW = 8


def _sc_mesh():
  return plsc.VectorSubcoreMesh(
      core_axis_name="c", subcore_axis_name="s", num_cores=NC,
      num_subcores=NS)


def _make_sc_agg(n_pad, n_chunks):
  rows_per_tile = n_pad // NS
  n_groups = n_chunks // GRP

  def body(y_hbm, cols_hbm, rows_hbm, out_hbm, col_v, row_v, bufs, acc,
           gsems, ssems):
    c = lax.axis_index("c")
    s = lax.axis_index("s")
    wid = c * NS + s

    zvec = jnp.zeros((L,), jnp.float32)

    @pl.loop(0, CHUNK)
    def _(i):
      bufs[0, 0, i, :] = zvec

    @pl.loop(0, rows_per_tile // CHUNK)
    def _(jz):
      pltpu.sync_copy(bufs.at[0, 0],
                      acc.at[pl.ds(s * rows_per_tile + jz * CHUNK, CHUNK)])

    pltpu.sync_copy(cols_hbm.at[wid], col_v)
    pltpu.sync_copy(rows_hbm.at[wid], row_v)

    plsc.subcore_barrier()

    def gather(i):
      p = i % 2
      for b in range(GRP):
        j = i * GRP + b
        pltpu.async_copy(y_hbm.at[col_v.at[j]], bufs.at[p, b], gsems[p])

    gather(0)
    for i in range(n_groups):
      p = i % 2
      if i + 1 < n_groups:
        if i >= 1:
          for b in range(GRP):
            pltpu.make_async_copy(bufs.at[1 - p, b],
                                  acc.at[row_v.at[b]], ssems[1 - p]).wait()
        gather(i + 1)
      for b in range(GRP):
        j = i * GRP + b
        pltpu.make_async_copy(y_hbm.at[col_v.at[j]], bufs.at[p, b],
                              gsems[p]).wait()
      for b in range(GRP):
        j = i * GRP + b
        pltpu.async_copy(bufs.at[p, b], acc.at[row_v.at[j]], ssems[p],
                         add=True)
    for i in (n_groups - 2, n_groups - 1):
      p = i % 2
      for b in range(GRP):
        pltpu.make_async_copy(bufs.at[p, b], acc.at[row_v.at[b]],
                              ssems[p]).wait()

    plsc.subcore_barrier()

    pltpu.sync_copy(
        acc.at[pl.ds(s * rows_per_tile, rows_per_tile)],
        out_hbm.at[c, pl.ds(s * rows_per_tile, rows_per_tile)])

  return pl.kernel(
      body,
      out_type=jax.ShapeDtypeStruct((NC, n_pad, F), jnp.float32),
      mesh=_sc_mesh(),
      compiler_params=pltpu.CompilerParams(use_tc_tiling_on_sc=False),
      scratch_types=[
          pltpu.VMEM((n_chunks, CHUNK), jnp.int32),
          pltpu.VMEM((n_chunks, CHUNK), jnp.int32),
          pltpu.VMEM((2, GRP, CHUNK, F), jnp.float32),
          pltpu.VMEM_SHARED((n_pad, F), jnp.float32),
          [pltpu.SemaphoreType.DMA, pltpu.SemaphoreType.DMA],
          [pltpu.SemaphoreType.DMA, pltpu.SemaphoreType.DMA],
      ],
  )


def _make_sc_deg(n_pad, n_chunks):
  rows_per_tile = n_pad // NS
  DEPTH = 8

  def body(rows_hbm, zeros_hbm, ones_hbm, out_hbm, row_v, ones_v, acc,
           ssem):
    c = lax.axis_index("c")
    s = lax.axis_index("s")
    wid = c * NS + s

    pltpu.sync_copy(zeros_hbm,
                    acc.at[pl.ds(s * rows_per_tile, rows_per_tile)])
    pltpu.sync_copy(ones_hbm, ones_v)
    pltpu.sync_copy(rows_hbm.at[wid], row_v)

    plsc.subcore_barrier()

    @pl.loop(0, n_chunks // DEPTH)
    def _(g):
      for b in range(DEPTH):
        pltpu.async_copy(ones_v, acc.at[row_v.at[g * DEPTH + b]], ssem,
                         add=True)
      for b in range(DEPTH):
        pltpu.make_async_copy(ones_v, acc.at[row_v.at[b]], ssem).wait()

    plsc.subcore_barrier()

    pltpu.sync_copy(
        acc.at[pl.ds(s * rows_per_tile, rows_per_tile)],
        out_hbm.at[c, pl.ds(s * rows_per_tile, rows_per_tile)])

  return pl.kernel(
      body,
      out_type=jax.ShapeDtypeStruct((NC, n_pad, DW), jnp.float32),
      mesh=_sc_mesh(),
      compiler_params=pltpu.CompilerParams(use_tc_tiling_on_sc=False),
      scratch_types=[
          pltpu.VMEM((n_chunks, CHUNK), jnp.int32),
          pltpu.VMEM((CHUNK, DW), jnp.float32),
          pltpu.VMEM_SHARED((n_pad, DW), jnp.float32),
          pltpu.SemaphoreType.DMA,
      ],
  )


_R = 2000


def _tc1a_body(x, w10, w11, xw0_o, xw1_o):
  xv = x[...]
  xw0_o[...] = jnp.dot(xv, w10[...], preferred_element_type=jnp.float32)
  xw1_o[...] = jnp.dot(xv, w11[...], preferred_element_type=jnp.float32)


def _tc1b_body(d0, xw1, y1_o, dis_o):
  deg = d0[...]
  dis8 = jnp.where(deg > 0.0, lax.rsqrt(jnp.where(deg > 0.0, deg, 1.0)),
                   0.0)
  dis = jnp.concatenate([dis8, dis8], axis=1)
  y1_o[...] = dis * xw1[...]
  dis_o[...] = dis


def _tc2_body(a0, xw0, dis, b1, w20, w21, hw0_o, y2_o):
  h = jnp.maximum(xw0[...] - dis[...] * a0[...] + b1[...], 0.0)
  hw0_o[...] = jnp.dot(h, w20[...], preferred_element_type=jnp.float32)
  y2_o[...] = dis[...] * jnp.dot(h, w21[...],
                                 preferred_element_type=jnp.float32)


def _tc3_body(a0, hw0, dis, b2, out_o):
  z = hw0[...] - dis[...] * a0[...] + b2[...]
  m = jnp.max(z, axis=1, keepdims=True)
  e = jnp.exp(z - m)
  out_o[...] = (z - m) - jnp.log(jnp.sum(e, axis=1, keepdims=True))


def _row_spec(w):
  return pl.BlockSpec((_R, w), lambda i: (i, 0))


def _full_spec(shape):
  return pl.BlockSpec(shape, lambda i: tuple(0 for _ in shape))


def kernel(x, edge_index, W1, b1, W2, b2):
  n, f_in = x.shape
  e = edge_index.shape[1]
  hid = W1.shape[2]
  c_out = W2.shape[2]

  per_w_chunks = -(-e // (NW * CHUNK))
  n_chunks = -(-per_w_chunks // (2 * GRP)) * (2 * GRP)
  e_pad = NW * n_chunks * CHUNK
  n_pad = -(-n // (NS * CHUNK)) * (NS * CHUNK)

  row = edge_index[0]
  col = edge_index[1]
  pad = jnp.full((e_pad - e,), n, jnp.int32)
  rows3 = jnp.concatenate([row, pad]).reshape(NW, n_chunks, CHUNK)
  cols3 = jnp.concatenate([col, pad]).reshape(NW, n_chunks, CHUNK)

  sc_deg = _make_sc_deg(n_pad, n_chunks)
  sc_agg = _make_sc_agg(n_pad, n_chunks)

  degp = sc_deg(rows3, jnp.zeros((n_pad // NS, DW), jnp.float32),
                jnp.ones((CHUNK, DW), jnp.float32))
  d0 = degp[0, :n, :]

  grid = (n // _R,)
  xw0, xw1 = pl.pallas_call(
      _tc1a_body,
      grid=grid,
      in_specs=[
          _row_spec(f_in),
          _full_spec((f_in, hid)),
          _full_spec((f_in, hid)),
      ],
      out_specs=[_row_spec(hid), _row_spec(hid)],
      out_shape=[
          jax.ShapeDtypeStruct((n, hid), jnp.float32),
          jax.ShapeDtypeStruct((n, hid), jnp.float32),
      ],
  )(x, W1[0], W1[1])

  y1, dis = pl.pallas_call(
      _tc1b_body,
      grid=grid,
      in_specs=[_row_spec(DW), _row_spec(hid)],
      out_specs=[_row_spec(hid), _row_spec(F)],
      out_shape=[
          jax.ShapeDtypeStruct((n, hid), jnp.float32),
          jax.ShapeDtypeStruct((n, F), jnp.float32),
      ],
  )(d0, xw1)

  y1p = jnp.pad(y1, ((0, n_pad - n), (0, 0)))
  acc1 = sc_agg(y1p, cols3, rows3)

  hw0, y2 = pl.pallas_call(
      _tc2_body,
      grid=grid,
      in_specs=[
          _row_spec(hid),
          _row_spec(hid),
          _row_spec(F),
          _full_spec((1, hid)),
          _full_spec((hid, c_out)),
          _full_spec((hid, c_out)),
      ],
      out_specs=[_row_spec(c_out), _row_spec(c_out)],
      out_shape=[
          jax.ShapeDtypeStruct((n, c_out), jnp.float32),
          jax.ShapeDtypeStruct((n, c_out), jnp.float32),
      ],
  )(acc1[0, :n, :], xw0, dis, b1.reshape(1, hid), W2[0], W2[1])

  y2p = jnp.pad(y2, ((0, n_pad - n), (0, 0)))
  acc2 = sc_agg(y2p, cols3, rows3)

  out = pl.pallas_call(
      _tc3_body,
      grid=grid,
      in_specs=[
          _row_spec(c_out),
          _row_spec(c_out),
          _row_spec(F),
          _full_spec((1, c_out)),
      ],
      out_specs=_row_spec(c_out),
      out_shape=jax.ShapeDtypeStruct((n, c_out), jnp.float32),
  )(acc2[0, :n, :], hw0, dis, b2.reshape(1, c_out))

  return out
